# baseline (speedup 1.0000x reference)
import jax
import jax.numpy as jnp
from jax import lax
from jax.experimental import pallas as pl
from jax.experimental.pallas import tpu as pltpu
from jax.experimental.pallas import tpu_sc as plsc

B = 16384
DIM = 32
NI = 5
NC = 2
NS = 16
NW = NC * NS
BPW = B // NW
CHUNK = 128
UCH = BPW // CHUNK
ICH = BPW * NI // CHUNK
GROUPS = BPW // 16


def _sc_body(uid_hbm, ids_hbm, utab_hbm, itab_hbm, s_hbm, d0_hbm,
             uidx_v, iidx_v, urows_v, irows_v, s_v, d0_v, sem):
    wid = lax.axis_index("s") * NC + lax.axis_index("c")
    base = wid * BPW

    pltpu.sync_copy(uid_hbm.at[pl.ds(base, BPW)], uidx_v)
    pltpu.sync_copy(ids_hbm.at[pl.ds(base * NI, BPW * NI)], iidx_v)

    copies = []
    for c in range(UCH):
        copies.append(pltpu.async_copy(
            utab_hbm.at[uidx_v.at[pl.ds(c * CHUNK, CHUNK)]],
            urows_v.at[pl.ds(c * CHUNK, CHUNK)], sem))
    for c in range(ICH):
        copies.append(pltpu.async_copy(
            itab_hbm.at[iidx_v.at[pl.ds(c * CHUNK, CHUNK)]],
            irows_v.at[pl.ds(c * CHUNK, CHUNK)], sem))
    for cp in copies:
        cp.wait()

    iota16 = lax.broadcasted_iota(jnp.int32, (16,), 0)
    perms = [jnp.bitwise_xor(iota16, o) for o in (1, 2, 4, 8)]
    masks = [(iota16 & o) == 0 for o in (1, 2, 4, 8)]

    def lane_sums(vregs):
        for st in range(4):
            perm, mask = perms[st], masks[st]
            nxt = []
            for k in range(len(vregs) // 2):
                a, b = vregs[2 * k], vregs[2 * k + 1]
                sa = a + a.at[perm].get(mode="promise_in_bounds")
                sb = b + b.at[perm].get(mode="promise_in_bounds")
                nxt.append(jnp.where(mask, sa, sb))
            vregs = nxt
        return vregs[0]

    def group(g, carry):
        u0 = []
        u1 = []
        for r in range(16):
            row = g * 16 + r
            u0.append(urows_v[row, pl.ds(0, 16)])
            u1.append(urows_v[row, pl.ds(16, 16)])
        dots = []
        for j in range(NI):
            prods = []
            for r in range(16):
                irow = (g * 16 + r) * NI + j
                i0 = irows_v[irow, pl.ds(0, 16)]
                i1 = irows_v[irow, pl.ds(16, 16)]
                prods.append(u0[r] * i0 + u1[r] * i1)
            dots.append(lane_sums(prods))
        ssum = jnp.exp(dots[0])
        for j in range(1, NI):
            ssum = ssum + jnp.exp(dots[j])
        s_v[pl.ds(g * 16, 16)] = ssum
        d0_v[pl.ds(g * 16, 16)] = dots[0]
        return carry

    lax.fori_loop(0, GROUPS, group, 0)

    pltpu.sync_copy(s_v, s_hbm.at[pl.ds(base, BPW)])
    pltpu.sync_copy(d0_v, d0_hbm.at[pl.ds(base, BPW)])


_sc_call = pl.kernel(
    _sc_body,
    mesh=plsc.VectorSubcoreMesh(core_axis_name="c", subcore_axis_name="s"),
    compiler_params=pltpu.CompilerParams(use_tc_tiling_on_sc=False),
    out_type=[
        jax.ShapeDtypeStruct((B,), jnp.float32),
        jax.ShapeDtypeStruct((B,), jnp.float32),
    ],
    scratch_types=[
        pltpu.VMEM((BPW,), jnp.int32),
        pltpu.VMEM((BPW * NI,), jnp.int32),
        pltpu.VMEM((BPW, DIM), jnp.float32),
        pltpu.VMEM((BPW * NI, DIM), jnp.float32),
        pltpu.VMEM((BPW,), jnp.float32),
        pltpu.VMEM((BPW,), jnp.float32),
        pltpu.SemaphoreType.DMA,
    ],
)


def _tc_loss_body(s_ref, d0_ref, out_ref):
    out_ref[0, 0] = (jnp.sum(jnp.log(s_ref[:])) - jnp.sum(d0_ref[:])) / B


_tc_loss = pl.pallas_call(
    _tc_loss_body,
    out_shape=jax.ShapeDtypeStruct((1, 1), jnp.float32),
    out_specs=pl.BlockSpec(memory_space=pltpu.SMEM),
)


def kernel(userid, itemid, user_feature, item_feature, neg_sample,
           user_table, item_table):
    uid = userid.reshape(B).astype(jnp.int32)
    ids = jnp.concatenate(
        [itemid.astype(jnp.int32), neg_sample.astype(jnp.int32)], axis=1
    ).reshape(B * NI)
    s, d0 = _sc_call(uid, ids, user_table, item_table)
    loss = _tc_loss(s.reshape(B // CHUNK, CHUNK), d0.reshape(B // CHUNK, CHUNK))
    return loss[0, 0]
